# baseline (speedup 1.0000x reference)
import jax
import jax.numpy as jnp
from jax.experimental import pallas as pl
from jax.experimental.pallas import tpu as pltpu

_N_CHUNKS = 8
_SEG = 128


def _body(srow_ref, scol_ref, ct_hbm, ind_ref, t_ref, out_hbm, indout_ref,
          seg_in, seg_out, *sems):
    row = srow_ref[0]
    col = scol_ref[0]
    n_det, n_cols = ct_hbm.shape
    chunk = n_cols // _N_CHUNKS
    base = pl.multiple_of((col // _SEG) * _SEG, _SEG)

    copies = [
        pltpu.make_async_copy(
            ct_hbm.at[:, pl.ds(i * chunk, chunk)],
            out_hbm.at[:, pl.ds(i * chunk, chunk)],
            sems[i],
        )
        for i in range(_N_CHUNKS)
    ]
    for c in copies:
        c.start()

    seg_load = pltpu.make_async_copy(
        ct_hbm.at[pl.ds(row, 1), pl.ds(base, _SEG)], seg_in, sems[_N_CHUNKS]
    )
    seg_load.start()

    lanes = jax.lax.broadcasted_iota(jnp.int32, ind_ref.shape, 1)
    indout_ref[...] = ind_ref[...] + (lanes == row).astype(jnp.int32)

    seg_load.wait()
    seg_lanes = jax.lax.broadcasted_iota(jnp.int32, (1, _SEG), 1) + base
    seg_out[...] = jnp.where(seg_lanes == col, t_ref[0, 0], seg_in[...])

    fixup = pltpu.make_async_copy(
        seg_out, out_hbm.at[pl.ds(row, 1), pl.ds(base, _SEG)], sems[_N_CHUNKS + 1]
    )
    cid = col // chunk
    for i, c in enumerate(copies):
        c.wait()

        @pl.when(cid == i)
        def _():
            fixup.start()

    fixup.wait()


def kernel(clicktimes, indices, idx, t):
    n_det, n_cols = clicktimes.shape
    row = jnp.asarray(idx, jnp.int32).reshape(1)
    col = jnp.take(indices, jnp.asarray(idx, jnp.int32)).reshape(1)
    ind2d = indices.reshape(1, n_det)
    t2d = jnp.asarray(t, jnp.float32).reshape(1, 1)

    out, indout = pl.pallas_call(
        _body,
        grid_spec=pltpu.PrefetchScalarGridSpec(
            num_scalar_prefetch=2,
            grid=(),
            in_specs=[
                pl.BlockSpec(memory_space=pltpu.HBM),
                pl.BlockSpec(memory_space=pltpu.VMEM),
                pl.BlockSpec(memory_space=pltpu.VMEM),
            ],
            out_specs=[
                pl.BlockSpec(memory_space=pltpu.HBM),
                pl.BlockSpec(memory_space=pltpu.VMEM),
            ],
            scratch_shapes=[
                pltpu.VMEM((1, _SEG), jnp.float32),
                pltpu.VMEM((1, _SEG), jnp.float32),
            ]
            + [pltpu.SemaphoreType.DMA] * (_N_CHUNKS + 2),
        ),
        out_shape=[
            jax.ShapeDtypeStruct((n_det, n_cols), clicktimes.dtype),
            jax.ShapeDtypeStruct((1, n_det), indices.dtype),
        ],
    )(row, col, clicktimes, ind2d, t2d)
    return (out, indout.reshape(n_det))


# speedup vs baseline: 1.0023x; 1.0023x over previous
import jax
import jax.numpy as jnp
from jax.experimental import pallas as pl
from jax.experimental.pallas import tpu as pltpu

_N_CHUNKS = 8
_SEG = 128


def _body(srow_ref, scol_ref, ct_hbm, ind_ref, t_ref, out_hbm, indout_ref,
          seg_in, seg_out, *sems):
    row = srow_ref[0]
    col = scol_ref[0]
    n_det, n_cols = ct_hbm.shape
    chunk = n_cols // _N_CHUNKS
    base = pl.multiple_of((col // _SEG) * _SEG, _SEG)

    rchunk = n_det // _N_CHUNKS
    copies = [
        pltpu.make_async_copy(
            ct_hbm.at[pl.ds(i * rchunk, rchunk), :],
            out_hbm.at[pl.ds(i * rchunk, rchunk), :],
            sems[i],
        )
        for i in range(_N_CHUNKS)
    ]
    for c in copies:
        c.start()

    seg_load = pltpu.make_async_copy(
        ct_hbm.at[pl.ds(row, 1), pl.ds(base, _SEG)], seg_in, sems[_N_CHUNKS]
    )
    seg_load.start()

    lanes = jax.lax.broadcasted_iota(jnp.int32, ind_ref.shape, 1)
    indout_ref[...] = ind_ref[...] + (lanes == row).astype(jnp.int32)

    seg_load.wait()
    seg_lanes = jax.lax.broadcasted_iota(jnp.int32, (1, _SEG), 1) + base
    seg_out[...] = jnp.where(seg_lanes == col, t_ref[0, 0], seg_in[...])

    fixup = pltpu.make_async_copy(
        seg_out, out_hbm.at[pl.ds(row, 1), pl.ds(base, _SEG)], sems[_N_CHUNKS + 1]
    )
    cid = row // rchunk
    for i, c in enumerate(copies):
        c.wait()

        @pl.when(cid == i)
        def _():
            fixup.start()

    fixup.wait()


def kernel(clicktimes, indices, idx, t):
    n_det, n_cols = clicktimes.shape
    row = jnp.asarray(idx, jnp.int32).reshape(1)
    col = jnp.take(indices, jnp.asarray(idx, jnp.int32)).reshape(1)
    ind2d = indices.reshape(1, n_det)
    t2d = jnp.asarray(t, jnp.float32).reshape(1, 1)

    out, indout = pl.pallas_call(
        _body,
        grid_spec=pltpu.PrefetchScalarGridSpec(
            num_scalar_prefetch=2,
            grid=(),
            in_specs=[
                pl.BlockSpec(memory_space=pltpu.HBM),
                pl.BlockSpec(memory_space=pltpu.VMEM),
                pl.BlockSpec(memory_space=pltpu.VMEM),
            ],
            out_specs=[
                pl.BlockSpec(memory_space=pltpu.HBM),
                pl.BlockSpec(memory_space=pltpu.VMEM),
            ],
            scratch_shapes=[
                pltpu.VMEM((1, _SEG), jnp.float32),
                pltpu.VMEM((1, _SEG), jnp.float32),
            ]
            + [pltpu.SemaphoreType.DMA] * (_N_CHUNKS + 2),
        ),
        out_shape=[
            jax.ShapeDtypeStruct((n_det, n_cols), clicktimes.dtype),
            jax.ShapeDtypeStruct((1, n_det), indices.dtype),
        ],
    )(row, col, clicktimes, ind2d, t2d)
    return (out, indout.reshape(n_det))


# speedup vs baseline: 29.6566x; 29.5881x over previous
import jax
import jax.numpy as jnp
from jax.experimental import pallas as pl
from jax.experimental.pallas import tpu as pltpu

_BLOCK_ROWS = 8


def _body(srow_ref, scol_ref, ct_ref, ind_ref, t_ref, out_ref, indout_ref):
    j = pl.program_id(0)
    row = srow_ref[0]
    col = scol_ref[0]
    base = j * _BLOCK_ROWS
    blk = ct_ref[...]
    hit = jnp.logical_and(row >= base, row < base + _BLOCK_ROWS)

    @pl.when(hit)
    def _():
        rows = jax.lax.broadcasted_iota(jnp.int32, blk.shape, 0) + base
        cols = jax.lax.broadcasted_iota(jnp.int32, blk.shape, 1)
        mask = jnp.logical_and(rows == row, cols == col)
        out_ref[...] = jnp.where(mask, t_ref[0, 0], blk)

    @pl.when(jnp.logical_not(hit))
    def _():
        out_ref[...] = blk

    @pl.when(j == 0)
    def _():
        lanes = jax.lax.broadcasted_iota(jnp.int32, ind_ref.shape, 1)
        indout_ref[...] = ind_ref[...] + (lanes == row).astype(jnp.int32)


def kernel(clicktimes, indices, idx, t):
    n_det, n_cols = clicktimes.shape
    grid = n_det // _BLOCK_ROWS
    row = jnp.asarray(idx, jnp.int32).reshape(1)
    col = jnp.take(indices, jnp.asarray(idx, jnp.int32)).reshape(1)
    ind2d = indices.reshape(1, n_det)
    t2d = jnp.asarray(t, jnp.float32).reshape(1, 1)

    out, indout = pl.pallas_call(
        _body,
        grid_spec=pltpu.PrefetchScalarGridSpec(
            num_scalar_prefetch=2,
            grid=(grid,),
            in_specs=[
                pl.BlockSpec((_BLOCK_ROWS, n_cols), lambda j, s1, s2: (j, 0)),
                pl.BlockSpec((1, n_det), lambda j, s1, s2: (0, 0)),
                pl.BlockSpec((1, 1), lambda j, s1, s2: (0, 0)),
            ],
            out_specs=[
                pl.BlockSpec((_BLOCK_ROWS, n_cols), lambda j, s1, s2: (j, 0)),
                pl.BlockSpec((1, n_det), lambda j, s1, s2: (0, 0)),
            ],
        ),
        out_shape=[
            jax.ShapeDtypeStruct((n_det, n_cols), clicktimes.dtype),
            jax.ShapeDtypeStruct((1, n_det), indices.dtype),
        ],
    )(row, col, clicktimes, ind2d, t2d)
    return (out, indout.reshape(n_det))


# speedup vs baseline: 30.3750x; 1.0242x over previous
import jax
import jax.numpy as jnp
from jax.experimental import pallas as pl
from jax.experimental.pallas import tpu as pltpu

_BLOCK_ROWS = 16


def _body(srow_ref, scol_ref, ct_ref, ind_ref, t_ref, out_ref, indout_ref):
    j = pl.program_id(0)
    row = srow_ref[0]
    col = scol_ref[0]
    base = j * _BLOCK_ROWS
    blk = ct_ref[...]
    hit = jnp.logical_and(row >= base, row < base + _BLOCK_ROWS)

    @pl.when(hit)
    def _():
        rows = jax.lax.broadcasted_iota(jnp.int32, blk.shape, 0) + base
        cols = jax.lax.broadcasted_iota(jnp.int32, blk.shape, 1)
        mask = jnp.logical_and(rows == row, cols == col)
        out_ref[...] = jnp.where(mask, t_ref[0, 0], blk)

    @pl.when(jnp.logical_not(hit))
    def _():
        out_ref[...] = blk

    @pl.when(j == 0)
    def _():
        lanes = jax.lax.broadcasted_iota(jnp.int32, ind_ref.shape, 1)
        indout_ref[...] = ind_ref[...] + (lanes == row).astype(jnp.int32)


def kernel(clicktimes, indices, idx, t):
    n_det, n_cols = clicktimes.shape
    grid = n_det // _BLOCK_ROWS
    row = jnp.asarray(idx, jnp.int32).reshape(1)
    col = jnp.take(indices, jnp.asarray(idx, jnp.int32)).reshape(1)
    ind2d = indices.reshape(1, n_det)
    t2d = jnp.asarray(t, jnp.float32).reshape(1, 1)

    out, indout = pl.pallas_call(
        _body,
        grid_spec=pltpu.PrefetchScalarGridSpec(
            num_scalar_prefetch=2,
            grid=(grid,),
            in_specs=[
                pl.BlockSpec((_BLOCK_ROWS, n_cols), lambda j, s1, s2: (j, 0)),
                pl.BlockSpec((1, n_det), lambda j, s1, s2: (0, 0)),
                pl.BlockSpec((1, 1), lambda j, s1, s2: (0, 0)),
            ],
            out_specs=[
                pl.BlockSpec((_BLOCK_ROWS, n_cols), lambda j, s1, s2: (j, 0)),
                pl.BlockSpec((1, n_det), lambda j, s1, s2: (0, 0)),
            ],
        ),
        out_shape=[
            jax.ShapeDtypeStruct((n_det, n_cols), clicktimes.dtype),
            jax.ShapeDtypeStruct((1, n_det), indices.dtype),
        ],
    )(row, col, clicktimes, ind2d, t2d)
    return (out, indout.reshape(n_det))
